# baseline (speedup 1.0000x reference)
import functools

import jax
import jax.numpy as jnp
from jax import lax
from jax.experimental import pallas as pl
from jax.experimental.pallas import tpu as pltpu
from jax.experimental.pallas import tpu_sc as plsc

NUM_ROAD = 1000000
D = 32
HALFD = 16
B = 16384
K = 20

NC = 2
NS = 16
NW = NC * NS
PER_W = B // NW
C = 64
NCHUNK = PER_W // C
CK = C * K
NSUB = CK // 128

def _dot(a0, a1, b0, b1):
    return jnp.sum(a0 * b0 + a1 * b1)


def _body(emb_1st, emb_2nd, emb_context, center, pos, negf,
          pos1, pos2, neg1p, neg2p,
          cen_v, pos_v, neg_v,
          e1_v, e2_v, ep1_v, ep2_v, en1_v, en2_v,
          o_p1, o_p2, o_n1, o_n2, sem):
    wid = lax.axis_index("s") * NC + lax.axis_index("c")
    base = wid * PER_W
    lane = lax.iota(jnp.int32, 16)

    pltpu.sync_copy(center.at[pl.ds(base, PER_W)], cen_v)
    pltpu.sync_copy(pos.at[pl.ds(base, PER_W)], pos_v)
    pltpu.sync_copy(negf.at[pl.ds(base * K, PER_W * K)], neg_v)

    def chunk_body(c, carry):
        co = c * C
        copies = []
        copies.append(pltpu.async_copy(
            emb_1st.at[cen_v.at[pl.ds(co, C)]], e1_v, sem))
        copies.append(pltpu.async_copy(
            emb_2nd.at[cen_v.at[pl.ds(co, C)]], e2_v, sem))
        copies.append(pltpu.async_copy(
            emb_1st.at[pos_v.at[pl.ds(co, C)]], ep1_v, sem))
        copies.append(pltpu.async_copy(
            emb_context.at[pos_v.at[pl.ds(co, C)]], ep2_v, sem))
        for j in range(NSUB):
            idx = neg_v.at[pl.ds(c * CK + j * 128, 128)]
            copies.append(pltpu.async_copy(
                emb_1st.at[idx], en1_v.at[pl.ds(j * 128, 128)], sem))
            copies.append(pltpu.async_copy(
                emb_context.at[idx], en2_v.at[pl.ds(j * 128, 128)], sem))
        for cp in copies:
            cp.wait()

        def pos_body(g, carry2):
            i0 = g * 16
            acc1 = jnp.zeros((16,), jnp.float32)
            acc2 = jnp.zeros((16,), jnp.float32)
            for j in range(16):
                i = i0 + j
                e1a = e1_v[i, pl.ds(0, HALFD)]
                e1b = e1_v[i, pl.ds(HALFD, HALFD)]
                ep1a = ep1_v[i, pl.ds(0, HALFD)]
                ep1b = ep1_v[i, pl.ds(HALFD, HALFD)]
                acc1 = jnp.where(lane == j, _dot(e1a, e1b, ep1a, ep1b), acc1)
                e2a = e2_v[i, pl.ds(0, HALFD)]
                e2b = e2_v[i, pl.ds(HALFD, HALFD)]
                ep2a = ep2_v[i, pl.ds(0, HALFD)]
                ep2b = ep2_v[i, pl.ds(HALFD, HALFD)]
                acc2 = jnp.where(lane == j, _dot(e2a, e2b, ep2a, ep2b), acc2)
            o_p1[pl.ds(i0, 16)] = acc1
            o_p2[pl.ds(i0, 16)] = acc2
            return carry2

        lax.fori_loop(0, C // 16, pos_body, 0, unroll=1)

        def neg_body(i, carry2):
            e1a = e1_v[i, pl.ds(0, HALFD)]
            e1b = e1_v[i, pl.ds(HALFD, HALFD)]
            e2a = e2_v[i, pl.ds(0, HALFD)]
            e2b = e2_v[i, pl.ds(HALFD, HALFD)]
            a10 = jnp.zeros((16,), jnp.float32)
            a11 = jnp.zeros((16,), jnp.float32)
            a20 = jnp.zeros((16,), jnp.float32)
            a21 = jnp.zeros((16,), jnp.float32)
            r0 = i * K
            for k in range(K):
                na = en1_v[r0 + k, pl.ds(0, HALFD)]
                nb = en1_v[r0 + k, pl.ds(HALFD, HALFD)]
                s1 = _dot(e1a, e1b, na, nb)
                ma = en2_v[r0 + k, pl.ds(0, HALFD)]
                mb = en2_v[r0 + k, pl.ds(HALFD, HALFD)]
                s2 = _dot(e2a, e2b, ma, mb)
                if k < 16:
                    a10 = jnp.where(lane == k, s1, a10)
                    a20 = jnp.where(lane == k, s2, a20)
                else:
                    a11 = jnp.where(lane == (k - 16), s1, a11)
                    a21 = jnp.where(lane == (k - 16), s2, a21)
            o_n1[i, pl.ds(0, 16)] = a10
            o_n1[i, pl.ds(16, 16)] = a11
            o_n2[i, pl.ds(0, 16)] = a20
            o_n2[i, pl.ds(16, 16)] = a21
            return carry2

        lax.fori_loop(0, C, neg_body, 0, unroll=1)

        pltpu.sync_copy(o_p1, pos1.at[pl.ds(base + co, C)])
        pltpu.sync_copy(o_p2, pos2.at[pl.ds(base + co, C)])
        pltpu.sync_copy(o_n1, neg1p.at[pl.ds(base + co, C)])
        pltpu.sync_copy(o_n2, neg2p.at[pl.ds(base + co, C)])
        return carry

    lax.fori_loop(0, NCHUNK, chunk_body, 0, unroll=1)


@functools.partial(
    pl.kernel,
    out_type=(
        jax.ShapeDtypeStruct((B,), jnp.float32),
        jax.ShapeDtypeStruct((B,), jnp.float32),
        jax.ShapeDtypeStruct((B, D), jnp.float32),
        jax.ShapeDtypeStruct((B, D), jnp.float32),
    ),
    mesh=plsc.VectorSubcoreMesh(core_axis_name="c", subcore_axis_name="s"),
    compiler_params=pltpu.CompilerParams(needs_layout_passes=False,
                                         use_tc_tiling_on_sc=False),
    scratch_types=[
        pltpu.VMEM((PER_W,), jnp.int32),
        pltpu.VMEM((PER_W,), jnp.int32),
        pltpu.VMEM((PER_W * K,), jnp.int32),
        pltpu.VMEM((C, D), jnp.float32),
        pltpu.VMEM((C, D), jnp.float32),
        pltpu.VMEM((C, D), jnp.float32),
        pltpu.VMEM((C, D), jnp.float32),
        pltpu.VMEM((CK, D), jnp.float32),
        pltpu.VMEM((CK, D), jnp.float32),
        pltpu.VMEM((C,), jnp.float32),
        pltpu.VMEM((C,), jnp.float32),
        pltpu.VMEM((C, D), jnp.float32),
        pltpu.VMEM((C, D), jnp.float32),
        pltpu.SemaphoreType.DMA,
    ],
)
def _sc_kernel(emb_1st, emb_2nd, emb_context, center, pos, negf,
               pos1, pos2, neg1p, neg2p, *scratch):
    _body(emb_1st, emb_2nd, emb_context, center, pos, negf,
          pos1, pos2, neg1p, neg2p, *scratch)


_TBW = 8192


def _transpose_body(i1, i2, i3, o1, o2, o3):
    o1[...] = i1[...].T
    o2[...] = i2[...].T
    o3[...] = i3[...].T


_tc_transpose = pl.pallas_call(
    _transpose_body,
    grid=(pl.cdiv(NUM_ROAD, _TBW),),
    in_specs=[pl.BlockSpec((D, _TBW), lambda i: (0, i))] * 3,
    out_specs=[pl.BlockSpec((_TBW, D), lambda i: (i, 0))] * 3,
    out_shape=[jax.ShapeDtypeStruct((NUM_ROAD, D), jnp.float32)] * 3,
)


def kernel(emb_1st, emb_2nd, emb_context, center, pos, neg):
    t1, t2, t3 = _tc_transpose(emb_1st.T, emb_2nd.T, emb_context.T)
    negf = neg.reshape(B * K)
    p1, p2, n1p, n2p = _sc_kernel(t1, t2, t3, center, pos, negf)
    return (p1, p2, n1p[:, :K], n2p[:, :K])


# speedup vs baseline: 2.0209x; 2.0209x over previous
import functools

import jax
import jax.numpy as jnp
from jax import lax
from jax.experimental import pallas as pl
from jax.experimental.pallas import tpu as pltpu
from jax.experimental.pallas import tpu_sc as plsc

NUM_ROAD = 1000000
D = 32
HALFD = 16
B = 16384
K = 20

NC = 2
NS = 16
NW = NC * NS
PER_W = B // NW
C = 64
NCHUNK = PER_W // C
CK = C * K
NSUB = CK // 128

_TBW = 8192
_T4 = _TBW // 4
_NBLK = pl.cdiv(NUM_ROAD, _TBW)
_NPACK = _NBLK * _TBW


def _transpose_body(i1, i2, i3, o1, o2, o3):
    for i_ref, o_ref in ((i1, o1), (i2, o2), (i3, o3)):
        x = i_ref[...]
        parts = [x[:, a * _T4:(a + 1) * _T4].T for a in range(4)]
        o_ref[...] = jnp.concatenate(parts, axis=1)


_tc_transpose = pl.pallas_call(
    _transpose_body,
    grid=(_NBLK,),
    in_specs=[pl.BlockSpec((D, _TBW), lambda i: (0, i))] * 3,
    out_specs=[pl.BlockSpec((_T4, 128), lambda i: (i, 0))] * 3,
    out_shape=[jax.ShapeDtypeStruct((_NBLK * _T4, 128), jnp.float32)] * 3,
)


def _permute_idx_ref(ref, n):
    def body(i, carry):
        v = ref[pl.ds(i * 16, 16)]
        sub = v & (_TBW - 1)
        ref[pl.ds(i * 16, 16)] = (v - sub) + 4 * (sub & (_T4 - 1)) + (
            sub >> 11)
        return carry

    lax.fori_loop(0, n // 16, body, 0, unroll=1)


def _dot(a0, a1, b0, b1):
    return jnp.sum(a0 * b0 + a1 * b1)


def _body(emb_1st, emb_2nd, emb_context, center, pos, negf,
          pos1, pos2, neg1p, neg2p,
          cen_v, pos_v, neg_v,
          e1_v, e2_v, ep1_v, ep2_v, en1_v, en2_v,
          o_p1, o_p2, o_n1, o_n2, sem):
    wid = lax.axis_index("s") * NC + lax.axis_index("c")
    base = wid * PER_W
    lane = lax.iota(jnp.int32, 16)

    pltpu.sync_copy(center.at[pl.ds(base, PER_W)], cen_v)
    pltpu.sync_copy(pos.at[pl.ds(base, PER_W)], pos_v)
    pltpu.sync_copy(negf.at[pl.ds(base * K, PER_W * K)], neg_v)
    _permute_idx_ref(cen_v, PER_W)
    _permute_idx_ref(pos_v, PER_W)
    _permute_idx_ref(neg_v, PER_W * K)

    def chunk_body(c, carry):
        co = c * C
        copies = []
        copies.append(pltpu.async_copy(
            emb_1st.at[cen_v.at[pl.ds(co, C)]], e1_v, sem))
        copies.append(pltpu.async_copy(
            emb_2nd.at[cen_v.at[pl.ds(co, C)]], e2_v, sem))
        copies.append(pltpu.async_copy(
            emb_1st.at[pos_v.at[pl.ds(co, C)]], ep1_v, sem))
        copies.append(pltpu.async_copy(
            emb_context.at[pos_v.at[pl.ds(co, C)]], ep2_v, sem))
        for j in range(NSUB):
            idx = neg_v.at[pl.ds(c * CK + j * 128, 128)]
            copies.append(pltpu.async_copy(
                emb_1st.at[idx], en1_v.at[pl.ds(j * 128, 128)], sem))
            copies.append(pltpu.async_copy(
                emb_context.at[idx], en2_v.at[pl.ds(j * 128, 128)], sem))
        for cp in copies:
            cp.wait()

        def pos_body(g, carry2):
            i0 = g * 16
            acc1 = jnp.zeros((16,), jnp.float32)
            acc2 = jnp.zeros((16,), jnp.float32)
            for j in range(16):
                i = i0 + j
                e1a = e1_v[i, pl.ds(0, HALFD)]
                e1b = e1_v[i, pl.ds(HALFD, HALFD)]
                ep1a = ep1_v[i, pl.ds(0, HALFD)]
                ep1b = ep1_v[i, pl.ds(HALFD, HALFD)]
                acc1 = jnp.where(lane == j, _dot(e1a, e1b, ep1a, ep1b), acc1)
                e2a = e2_v[i, pl.ds(0, HALFD)]
                e2b = e2_v[i, pl.ds(HALFD, HALFD)]
                ep2a = ep2_v[i, pl.ds(0, HALFD)]
                ep2b = ep2_v[i, pl.ds(HALFD, HALFD)]
                acc2 = jnp.where(lane == j, _dot(e2a, e2b, ep2a, ep2b), acc2)
            o_p1[pl.ds(i0, 16)] = acc1
            o_p2[pl.ds(i0, 16)] = acc2
            return carry2

        lax.fori_loop(0, C // 16, pos_body, 0, unroll=1)

        def neg_body(i, carry2):
            e1a = e1_v[i, pl.ds(0, HALFD)]
            e1b = e1_v[i, pl.ds(HALFD, HALFD)]
            e2a = e2_v[i, pl.ds(0, HALFD)]
            e2b = e2_v[i, pl.ds(HALFD, HALFD)]
            a10 = jnp.zeros((16,), jnp.float32)
            a11 = jnp.zeros((16,), jnp.float32)
            a20 = jnp.zeros((16,), jnp.float32)
            a21 = jnp.zeros((16,), jnp.float32)
            r0 = i * K
            for k in range(K):
                na = en1_v[r0 + k, pl.ds(0, HALFD)]
                nb = en1_v[r0 + k, pl.ds(HALFD, HALFD)]
                s1 = _dot(e1a, e1b, na, nb)
                ma = en2_v[r0 + k, pl.ds(0, HALFD)]
                mb = en2_v[r0 + k, pl.ds(HALFD, HALFD)]
                s2 = _dot(e2a, e2b, ma, mb)
                if k < 16:
                    a10 = jnp.where(lane == k, s1, a10)
                    a20 = jnp.where(lane == k, s2, a20)
                else:
                    a11 = jnp.where(lane == (k - 16), s1, a11)
                    a21 = jnp.where(lane == (k - 16), s2, a21)
            o_n1[i, pl.ds(0, 16)] = a10
            o_n1[i, pl.ds(16, 16)] = a11
            o_n2[i, pl.ds(0, 16)] = a20
            o_n2[i, pl.ds(16, 16)] = a21
            return carry2

        lax.fori_loop(0, C, neg_body, 0, unroll=1)

        pltpu.sync_copy(o_p1, pos1.at[pl.ds(base + co, C)])
        pltpu.sync_copy(o_p2, pos2.at[pl.ds(base + co, C)])
        pltpu.sync_copy(o_n1, neg1p.at[pl.ds(base + co, C)])
        pltpu.sync_copy(o_n2, neg2p.at[pl.ds(base + co, C)])
        return carry

    lax.fori_loop(0, NCHUNK, chunk_body, 0, unroll=1)


@functools.partial(
    pl.kernel,
    out_type=(
        jax.ShapeDtypeStruct((B,), jnp.float32),
        jax.ShapeDtypeStruct((B,), jnp.float32),
        jax.ShapeDtypeStruct((B, D), jnp.float32),
        jax.ShapeDtypeStruct((B, D), jnp.float32),
    ),
    mesh=plsc.VectorSubcoreMesh(core_axis_name="c", subcore_axis_name="s"),
    compiler_params=pltpu.CompilerParams(needs_layout_passes=False,
                                         use_tc_tiling_on_sc=False),
    scratch_types=[
        pltpu.VMEM((PER_W,), jnp.int32),
        pltpu.VMEM((PER_W,), jnp.int32),
        pltpu.VMEM((PER_W * K,), jnp.int32),
        pltpu.VMEM((C, D), jnp.float32),
        pltpu.VMEM((C, D), jnp.float32),
        pltpu.VMEM((C, D), jnp.float32),
        pltpu.VMEM((C, D), jnp.float32),
        pltpu.VMEM((CK, D), jnp.float32),
        pltpu.VMEM((CK, D), jnp.float32),
        pltpu.VMEM((C,), jnp.float32),
        pltpu.VMEM((C,), jnp.float32),
        pltpu.VMEM((C, D), jnp.float32),
        pltpu.VMEM((C, D), jnp.float32),
        pltpu.SemaphoreType.DMA,
    ],
)
def _sc_kernel(emb_1st, emb_2nd, emb_context, center, pos, negf,
               pos1, pos2, neg1p, neg2p, *scratch):
    _body(emb_1st, emb_2nd, emb_context, center, pos, negf,
          pos1, pos2, neg1p, neg2p, *scratch)


def kernel(emb_1st, emb_2nd, emb_context, center, pos, neg):
    t1, t2, t3 = _tc_transpose(emb_1st.T, emb_2nd.T, emb_context.T)
    t1 = t1.reshape(_NPACK, D)
    t2 = t2.reshape(_NPACK, D)
    t3 = t3.reshape(_NPACK, D)
    negf = neg.reshape(B * K)
    p1, p2, n1p, n2p = _sc_kernel(t1, t2, t3, center, pos, negf)
    return (p1, p2, n1p[:, :K], n2p[:, :K])


# speedup vs baseline: 3.0047x; 1.4868x over previous
import functools

import jax
import jax.numpy as jnp
from jax import lax
from jax.experimental import pallas as pl
from jax.experimental.pallas import tpu as pltpu
from jax.experimental.pallas import tpu_sc as plsc

NUM_ROAD = 1000000
D = 32
HALFD = 16
B = 16384
K = 20

NC = 2
NS = 16
NW = NC * NS
PER_W = B // NW
C = 64
NCHUNK = PER_W // C
CK = C * K
NSUB = CK // 128

_TBW = 8192
_T4 = _TBW // 4
_NBLK = pl.cdiv(NUM_ROAD, _TBW)
_NPACK = _NBLK * _TBW


def _transpose_body(i1, i2, i3, o1, o2, o3):
    eye = jnp.eye(D, dtype=jnp.float32)
    dn = (((0,), (0,)), ((), ()))
    for i_ref, o_ref in ((i1, o1), (i2, o2), (i3, o3)):
        acc = None
        for a in range(4):
            ea = jnp.pad(eye, ((0, 0), (a * D, 128 - (a + 1) * D)))
            y = jax.lax.dot_general(i_ref[:, pl.ds(a * _T4, _T4)], ea, dn,
                                    preferred_element_type=jnp.float32)
            acc = y if acc is None else acc + y
        o_ref[...] = acc


_tc_transpose = pl.pallas_call(
    _transpose_body,
    grid=(_NBLK,),
    in_specs=[pl.BlockSpec((D, _TBW), lambda i: (0, i))] * 3,
    out_specs=[pl.BlockSpec((_T4, 128), lambda i: (i, 0))] * 3,
    out_shape=[jax.ShapeDtypeStruct((_NBLK * _T4, 128), jnp.float32)] * 3,
)


def _permute_idx_ref(ref, n):
    def body(i, carry):
        v = ref[pl.ds(i * 16, 16)]
        sub = v & (_TBW - 1)
        ref[pl.ds(i * 16, 16)] = (v - sub) + 4 * (sub & (_T4 - 1)) + (
            sub >> 11)
        return carry

    lax.fori_loop(0, n // 16, body, 0, unroll=1)


def _dot(a0, a1, b0, b1):
    return jnp.sum(a0 * b0 + a1 * b1)


def _body(emb_1st, emb_2nd, emb_context, center, pos, negf,
          pos1, pos2, neg1p, neg2p,
          cen_v, pos_v, neg_v,
          e1_v, e2_v, ep1_v, ep2_v, en1_v, en2_v,
          o_p1, o_p2, o_n1, o_n2, sem):
    wid = lax.axis_index("s") * NC + lax.axis_index("c")
    base = wid * PER_W
    lane = lax.iota(jnp.int32, 16)

    pltpu.sync_copy(center.at[pl.ds(base, PER_W)], cen_v)
    pltpu.sync_copy(pos.at[pl.ds(base, PER_W)], pos_v)
    pltpu.sync_copy(negf.at[pl.ds(base * K, PER_W * K)], neg_v)
    _permute_idx_ref(cen_v, PER_W)
    _permute_idx_ref(pos_v, PER_W)
    _permute_idx_ref(neg_v, PER_W * K)

    def chunk_body(c, carry):
        co = c * C
        copies = []
        copies.append(pltpu.async_copy(
            emb_1st.at[cen_v.at[pl.ds(co, C)]], e1_v, sem))
        copies.append(pltpu.async_copy(
            emb_2nd.at[cen_v.at[pl.ds(co, C)]], e2_v, sem))
        copies.append(pltpu.async_copy(
            emb_1st.at[pos_v.at[pl.ds(co, C)]], ep1_v, sem))
        copies.append(pltpu.async_copy(
            emb_context.at[pos_v.at[pl.ds(co, C)]], ep2_v, sem))
        for j in range(NSUB):
            idx = neg_v.at[pl.ds(c * CK + j * 128, 128)]
            copies.append(pltpu.async_copy(
                emb_1st.at[idx], en1_v.at[pl.ds(j * 128, 128)], sem))
            copies.append(pltpu.async_copy(
                emb_context.at[idx], en2_v.at[pl.ds(j * 128, 128)], sem))
        for cp in copies:
            cp.wait()

        def pos_body(g, carry2):
            i0 = g * 16
            acc1 = jnp.zeros((16,), jnp.float32)
            acc2 = jnp.zeros((16,), jnp.float32)
            for j in range(16):
                i = i0 + j
                e1a = e1_v[i, pl.ds(0, HALFD)]
                e1b = e1_v[i, pl.ds(HALFD, HALFD)]
                ep1a = ep1_v[i, pl.ds(0, HALFD)]
                ep1b = ep1_v[i, pl.ds(HALFD, HALFD)]
                acc1 = jnp.where(lane == j, _dot(e1a, e1b, ep1a, ep1b), acc1)
                e2a = e2_v[i, pl.ds(0, HALFD)]
                e2b = e2_v[i, pl.ds(HALFD, HALFD)]
                ep2a = ep2_v[i, pl.ds(0, HALFD)]
                ep2b = ep2_v[i, pl.ds(HALFD, HALFD)]
                acc2 = jnp.where(lane == j, _dot(e2a, e2b, ep2a, ep2b), acc2)
            o_p1[pl.ds(i0, 16)] = acc1
            o_p2[pl.ds(i0, 16)] = acc2
            return carry2

        lax.fori_loop(0, C // 16, pos_body, 0, unroll=1)

        def neg_body(i, carry2):
            e1a = e1_v[i, pl.ds(0, HALFD)]
            e1b = e1_v[i, pl.ds(HALFD, HALFD)]
            e2a = e2_v[i, pl.ds(0, HALFD)]
            e2b = e2_v[i, pl.ds(HALFD, HALFD)]
            a10 = jnp.zeros((16,), jnp.float32)
            a11 = jnp.zeros((16,), jnp.float32)
            a20 = jnp.zeros((16,), jnp.float32)
            a21 = jnp.zeros((16,), jnp.float32)
            r0 = i * K
            for k in range(K):
                na = en1_v[r0 + k, pl.ds(0, HALFD)]
                nb = en1_v[r0 + k, pl.ds(HALFD, HALFD)]
                s1 = _dot(e1a, e1b, na, nb)
                ma = en2_v[r0 + k, pl.ds(0, HALFD)]
                mb = en2_v[r0 + k, pl.ds(HALFD, HALFD)]
                s2 = _dot(e2a, e2b, ma, mb)
                if k < 16:
                    a10 = jnp.where(lane == k, s1, a10)
                    a20 = jnp.where(lane == k, s2, a20)
                else:
                    a11 = jnp.where(lane == (k - 16), s1, a11)
                    a21 = jnp.where(lane == (k - 16), s2, a21)
            o_n1[i, pl.ds(0, 16)] = a10
            o_n1[i, pl.ds(16, 16)] = a11
            o_n2[i, pl.ds(0, 16)] = a20
            o_n2[i, pl.ds(16, 16)] = a21
            return carry2

        lax.fori_loop(0, C, neg_body, 0, unroll=1)

        pltpu.sync_copy(o_p1, pos1.at[pl.ds(base + co, C)])
        pltpu.sync_copy(o_p2, pos2.at[pl.ds(base + co, C)])
        pltpu.sync_copy(o_n1, neg1p.at[pl.ds(base + co, C)])
        pltpu.sync_copy(o_n2, neg2p.at[pl.ds(base + co, C)])
        return carry

    lax.fori_loop(0, NCHUNK, chunk_body, 0, unroll=1)


@functools.partial(
    pl.kernel,
    out_type=(
        jax.ShapeDtypeStruct((B,), jnp.float32),
        jax.ShapeDtypeStruct((B,), jnp.float32),
        jax.ShapeDtypeStruct((B, D), jnp.float32),
        jax.ShapeDtypeStruct((B, D), jnp.float32),
    ),
    mesh=plsc.VectorSubcoreMesh(core_axis_name="c", subcore_axis_name="s"),
    compiler_params=pltpu.CompilerParams(needs_layout_passes=False,
                                         use_tc_tiling_on_sc=False),
    scratch_types=[
        pltpu.VMEM((PER_W,), jnp.int32),
        pltpu.VMEM((PER_W,), jnp.int32),
        pltpu.VMEM((PER_W * K,), jnp.int32),
        pltpu.VMEM((C, D), jnp.float32),
        pltpu.VMEM((C, D), jnp.float32),
        pltpu.VMEM((C, D), jnp.float32),
        pltpu.VMEM((C, D), jnp.float32),
        pltpu.VMEM((CK, D), jnp.float32),
        pltpu.VMEM((CK, D), jnp.float32),
        pltpu.VMEM((C,), jnp.float32),
        pltpu.VMEM((C,), jnp.float32),
        pltpu.VMEM((C, D), jnp.float32),
        pltpu.VMEM((C, D), jnp.float32),
        pltpu.SemaphoreType.DMA,
    ],
)
def _sc_kernel(emb_1st, emb_2nd, emb_context, center, pos, negf,
               pos1, pos2, neg1p, neg2p, *scratch):
    _body(emb_1st, emb_2nd, emb_context, center, pos, negf,
          pos1, pos2, neg1p, neg2p, *scratch)


def kernel(emb_1st, emb_2nd, emb_context, center, pos, neg):
    t1, t2, t3 = _tc_transpose(emb_1st.T, emb_2nd.T, emb_context.T)
    t1 = t1.reshape(_NPACK, D)
    t2 = t2.reshape(_NPACK, D)
    t3 = t3.reshape(_NPACK, D)
    negf = neg.reshape(B * K)
    p1, p2, n1p, n2p = _sc_kernel(t1, t2, t3, center, pos, negf)
    return (p1, p2, n1p[:, :K], n2p[:, :K])
